# baseline (speedup 1.0000x reference)
import functools

import jax
import jax.numpy as jnp
from jax.experimental import pallas as pl
from jax.experimental.pallas import tpu as pltpu
from jax.experimental.pallas import tpu_sc as plsc

VOCAB = 100000
D = 128
B = 4096
CTX = 20

_SC_ELEMS = 8
_SC_ROWS = _SC_ELEMS * CTX
_SC_HALF = _SC_ROWS // 2
_LANES = 16


def _sc_gather_sum(emb_table, idx_flat):
  mesh = plsc.VectorSubcoreMesh(core_axis_name="core", subcore_axis_name="subcore")

  @functools.partial(
      pl.kernel,
      out_type=jax.ShapeDtypeStruct((B, D), jnp.float32),
      mesh=mesh,
      scratch_types=[
          pltpu.VMEM((_SC_ROWS, D), jnp.float32),
          pltpu.SemaphoreType.DMA,
          pltpu.SemaphoreType.DMA,
      ],
  )
  def sc_kernel(emb_hbm, idx_hbm, out_hbm, rows_vmem, sem0, sem1):
    def body(idx_vmem, out_vmem):
      cp0 = pltpu.async_copy(
          emb_hbm.at[idx_vmem.at[pl.ds(0, _SC_HALF)]],
          rows_vmem.at[pl.ds(0, _SC_HALF)], sem0)
      cp1 = pltpu.async_copy(
          emb_hbm.at[idx_vmem.at[pl.ds(_SC_HALF, _SC_HALF)]],
          rows_vmem.at[pl.ds(_SC_HALF, _SC_HALF)], sem1)
      cp0.wait()
      cp1.wait()
      for e in range(_SC_ELEMS):
        for l in range(D // _LANES):
          sl = pl.ds(l * _LANES, _LANES)
          acc = rows_vmem.at[pl.ds(e * CTX, 1), sl][...]
          for c in range(1, CTX):
            acc = acc + rows_vmem.at[pl.ds(e * CTX + c, 1), sl][...]
          out_vmem.at[pl.ds(e, 1), sl][...] = acc

    pltpu.emit_pipeline(
        body,
        grid=(B // _SC_ELEMS,),
        in_specs=[pl.BlockSpec((_SC_ROWS,), index_map=lambda i: (i,))],
        out_specs=[pl.BlockSpec((_SC_ELEMS, D), index_map=lambda i: (i, 0))],
        core_axis_name=("core", "subcore"),
        dimension_semantics=(pltpu.PARALLEL,),
    )(idx_hbm, out_hbm)

  return sc_kernel(emb_table, idx_flat)


_BV = 1000
_NV = VOCAB // _BV


def _mm_body(x_ref, w_ref, b_ref, o_ref, xs_ref):
  @pl.when(pl.program_id(0) == 0)
  def _():
    xs_ref[...] = (x_ref[...] * (1.0 / CTX)).astype(jnp.bfloat16)

  acc = jax.lax.dot_general(
      w_ref[...], xs_ref[...], (((1,), (1,)), ((), ())),
      preferred_element_type=jnp.float32)
  o_ref[...] = acc + b_ref[...]


def _tc_project(ctx_sum, w_bf16, bias_col):
  grid = (_NV,)
  out_t = pl.pallas_call(
      _mm_body,
      grid=grid,
      in_specs=[
          pl.BlockSpec((B, D), lambda j: (0, 0)),
          pl.BlockSpec((_BV, D), lambda j: (j, 0)),
          pl.BlockSpec((_BV, 1), lambda j: (j, 0)),
      ],
      out_specs=pl.BlockSpec((_BV, B), lambda j: (j, 0)),
      out_shape=jax.ShapeDtypeStruct((VOCAB, B), jnp.float32),
      scratch_shapes=[pltpu.VMEM((B, D), jnp.bfloat16)],
      compiler_params=pltpu.CompilerParams(
          dimension_semantics=("arbitrary",)),
  )(ctx_sum, w_bf16, bias_col)
  return jnp.transpose(out_t)


def kernel(inputs, emb_table, lin_w, lin_b):
  idx_flat = inputs.astype(jnp.int32).reshape(B * CTX)
  ctx_sum = _sc_gather_sum(emb_table, idx_flat)
  w_bf16 = lin_w.astype(jnp.bfloat16)
  bias_col = lin_b.reshape(VOCAB, 1)
  return _tc_project(ctx_sum, w_bf16, bias_col)


# speedup vs baseline: 1.0512x; 1.0512x over previous
import functools

import jax
import jax.numpy as jnp
from jax.experimental import pallas as pl
from jax.experimental.pallas import tpu as pltpu
from jax.experimental.pallas import tpu_sc as plsc

VOCAB = 100000
D = 128
B = 4096
CTX = 20

_SC_ELEMS = 4
_SC_ROWS = _SC_ELEMS * CTX
_LANES = 16


def _sc_gather_sum(emb_table, idx_flat):
  mesh = plsc.VectorSubcoreMesh(core_axis_name="core", subcore_axis_name="subcore")

  @functools.partial(
      pl.kernel,
      out_type=jax.ShapeDtypeStruct((B, D), jnp.float32),
      mesh=mesh,
      scratch_types=[pltpu.VMEM((_SC_ROWS, D), jnp.float32)],
  )
  def sc_kernel(emb_hbm, idx_hbm, out_hbm, rows_vmem):
    def body(idx_vmem, out_vmem):
      pltpu.sync_copy(emb_hbm.at[idx_vmem], rows_vmem)
      for e in range(_SC_ELEMS):
        for l in range(D // _LANES):
          sl = pl.ds(l * _LANES, _LANES)
          acc = rows_vmem.at[pl.ds(e * CTX, 1), sl][...]
          for c in range(1, CTX):
            acc = acc + rows_vmem.at[pl.ds(e * CTX + c, 1), sl][...]
          out_vmem.at[pl.ds(e, 1), sl][...] = acc

    pltpu.emit_pipeline(
        body,
        grid=(B // _SC_ELEMS,),
        in_specs=[pl.BlockSpec((_SC_ROWS,), index_map=lambda i: (i,))],
        out_specs=[pl.BlockSpec((_SC_ELEMS, D), index_map=lambda i: (i, 0))],
        core_axis_name=("core", "subcore"),
        dimension_semantics=(pltpu.PARALLEL,),
    )(idx_hbm, out_hbm)

  return sc_kernel(emb_table, idx_flat)


_BV = 1000
_NV = VOCAB // _BV


def _mm_body(x_ref, w_ref, b_ref, o_ref, xs_ref):
  @pl.when(pl.program_id(0) == 0)
  def _():
    xs_ref[...] = (x_ref[...] * (1.0 / CTX)).astype(jnp.bfloat16)

  acc = jax.lax.dot_general(
      w_ref[...], xs_ref[...], (((1,), (1,)), ((), ())),
      preferred_element_type=jnp.float32)
  o_ref[...] = acc + b_ref[...]


def _tc_project(ctx_sum, w_bf16, bias_col):
  grid = (_NV,)
  out_t = pl.pallas_call(
      _mm_body,
      grid=grid,
      in_specs=[
          pl.BlockSpec((B, D), lambda j: (0, 0)),
          pl.BlockSpec((_BV, D), lambda j: (j, 0)),
          pl.BlockSpec((_BV, 1), lambda j: (j, 0)),
      ],
      out_specs=pl.BlockSpec((_BV, B), lambda j: (j, 0)),
      out_shape=jax.ShapeDtypeStruct((VOCAB, B), jnp.float32),
      scratch_shapes=[pltpu.VMEM((B, D), jnp.bfloat16)],
      compiler_params=pltpu.CompilerParams(
          dimension_semantics=("arbitrary",)),
  )(ctx_sum, w_bf16, bias_col)
  return jnp.transpose(out_t)


def kernel(inputs, emb_table, lin_w, lin_b):
  idx_flat = inputs.astype(jnp.int32).reshape(B * CTX)
  ctx_sum = _sc_gather_sum(emb_table, idx_flat)
  w_bf16 = lin_w.astype(jnp.bfloat16)
  bias_col = lin_b.reshape(VOCAB, 1)
  return _tc_project(ctx_sum, w_bf16, bias_col)
